# baseline (speedup 1.0000x reference)
import jax
import jax.numpy as jnp
from jax.experimental import pallas as pl
from jax.experimental.pallas import tpu as pltpu

N = 8192
BM = 256


def _body(xc0_ref, xc1_ref, xt0_ref, xt1_ref, out_ref, colmin_ref, rowsum_ref):
    i = pl.program_id(0)

    @pl.when(i == 0)
    def _init():
        colmin_ref[...] = jnp.full((1, N), jnp.inf, dtype=jnp.float32)
        rowsum_ref[0] = 0.0

    xc0 = xc0_ref[...]
    xc1 = xc1_ref[...]
    xt0 = xt0_ref[...]
    xt1 = xt1_ref[...]

    nc = xc0 * xc0 + xc1 * xc1
    nt = xt0 * xt0 + xt1 * xt1

    a0 = xc0.astype(jnp.bfloat16).astype(jnp.float32)
    a1 = xc1.astype(jnp.bfloat16).astype(jnp.float32)
    b0 = xt0.astype(jnp.bfloat16).astype(jnp.float32)
    b1 = xt1.astype(jnp.bfloat16).astype(jnp.float32)

    cross = a0 * b0 + a1 * b1
    s = jnp.maximum((nc + nt) - 2.0 * cross, 0.0)

    rowsum_ref[0] += jnp.sum(jnp.min(s, axis=1))
    colmin_ref[...] = jnp.minimum(colmin_ref[...], jnp.min(s, axis=0, keepdims=True))

    @pl.when(i == pl.num_programs(0) - 1)
    def _fin():
        out_ref[0, 0] = (rowsum_ref[0] + jnp.sum(colmin_ref[...])) / N


def kernel(Xc, Xt):
    xc0 = Xc[:, 0:1]
    xc1 = Xc[:, 1:2]
    xt0 = Xt[:, 0].reshape(1, N)
    xt1 = Xt[:, 1].reshape(1, N)
    out = pl.pallas_call(
        _body,
        grid=(N // BM,),
        in_specs=[
            pl.BlockSpec((BM, 1), lambda i: (i, 0)),
            pl.BlockSpec((BM, 1), lambda i: (i, 0)),
            pl.BlockSpec((1, N), lambda i: (0, 0)),
            pl.BlockSpec((1, N), lambda i: (0, 0)),
        ],
        out_specs=pl.BlockSpec((1, 1), lambda i: (0, 0), memory_space=pltpu.SMEM),
        out_shape=jax.ShapeDtypeStruct((1, 1), jnp.float32),
        scratch_shapes=[
            pltpu.VMEM((1, N), jnp.float32),
            pltpu.SMEM((1,), jnp.float32),
        ],
    )(xc0, xc1, xt0, xt1)
    return out[0, 0]


# speedup vs baseline: 1.3114x; 1.3114x over previous
import jax
import jax.numpy as jnp
from jax.experimental import pallas as pl
from jax.experimental.pallas import tpu as pltpu

N = 8192
BM = 256


def _body(xc0_ref, xc1_ref, xt0_ref, xt1_ref, out_ref, colmin_ref, rowsum_ref):
    i = pl.program_id(0)

    @pl.when(i == 0)
    def _init():
        colmin_ref[...] = jnp.full((1, N), jnp.inf, dtype=jnp.float32)
        rowsum_ref[0] = 0.0

    xc0 = xc0_ref[...]
    xc1 = xc1_ref[...]
    xt0 = xt0_ref[...]
    xt1 = xt1_ref[...]

    nc = xc0 * xc0 + xc1 * xc1
    nt = xt0 * xt0 + xt1 * xt1

    a0s = -2.0 * xc0.astype(jnp.bfloat16).astype(jnp.float32)
    a1s = -2.0 * xc1.astype(jnp.bfloat16).astype(jnp.float32)
    b0 = xt0.astype(jnp.bfloat16).astype(jnp.float32)
    b1 = xt1.astype(jnp.bfloat16).astype(jnp.float32)

    c2n = a0s * b0 + a1s * b1
    t = c2n + nt
    u = c2n + nc

    rmin = jnp.min(t, axis=1, keepdims=True)
    rowsum_ref[0] += jnp.sum(jnp.maximum(nc + rmin, 0.0))
    colmin_ref[...] = jnp.minimum(colmin_ref[...], jnp.min(u, axis=0, keepdims=True))

    @pl.when(i == pl.num_programs(0) - 1)
    def _fin():
        colfin = jnp.maximum(nt + colmin_ref[...], 0.0)
        out_ref[0, 0] = (rowsum_ref[0] + jnp.sum(colfin)) / N


def kernel(Xc, Xt):
    xc0 = Xc[:, 0:1]
    xc1 = Xc[:, 1:2]
    xt0 = Xt[:, 0].reshape(1, N)
    xt1 = Xt[:, 1].reshape(1, N)
    out = pl.pallas_call(
        _body,
        grid=(N // BM,),
        in_specs=[
            pl.BlockSpec((BM, 1), lambda i: (i, 0)),
            pl.BlockSpec((BM, 1), lambda i: (i, 0)),
            pl.BlockSpec((1, N), lambda i: (0, 0)),
            pl.BlockSpec((1, N), lambda i: (0, 0)),
        ],
        out_specs=pl.BlockSpec((1, 1), lambda i: (0, 0), memory_space=pltpu.SMEM),
        out_shape=jax.ShapeDtypeStruct((1, 1), jnp.float32),
        scratch_shapes=[
            pltpu.VMEM((1, N), jnp.float32),
            pltpu.SMEM((1,), jnp.float32),
        ],
    )(xc0, xc1, xt0, xt1)
    return out[0, 0]


# speedup vs baseline: 1.7529x; 1.3367x over previous
import jax
import jax.numpy as jnp
from jax.experimental import pallas as pl
from jax.experimental.pallas import tpu as pltpu

N = 8192
BM = 512


def _split3(x):
    mask = jnp.uint32(0xFFFF0000)

    def trunc(v):
        return jax.lax.bitcast_convert_type(
            jax.lax.bitcast_convert_type(v, jnp.uint32) & mask, jnp.float32)

    m1 = trunc(x)
    r1 = x - m1
    m2 = trunc(r1)
    r2 = r1 - m2
    m3 = trunc(r2)
    return m1, m2, m3


def _body(a_ref, b_ref, xc0_ref, xc1_ref, out_ref, colmin_ref, rowsum_ref):
    i = pl.program_id(0)

    @pl.when(i == 0)
    def _init():
        colmin_ref[...] = jnp.full((1, N), jnp.inf, dtype=jnp.float32)
        rowsum_ref[0] = 0.0

    T = jax.lax.dot_general(
        a_ref[...], b_ref[...],
        dimension_numbers=(((1,), (0,)), ((), ())),
        preferred_element_type=jnp.float32,
    )

    xc0 = xc0_ref[...]
    xc1 = xc1_ref[...]
    nc = xc0 * xc0 + xc1 * xc1

    rmin = jnp.min(T, axis=1, keepdims=True)
    rowsum_ref[0] += jnp.sum(jnp.maximum(nc + rmin, 0.0))
    v = T + nc
    colmin_ref[...] = jnp.minimum(colmin_ref[...], jnp.min(v, axis=0, keepdims=True))

    @pl.when(i == pl.num_programs(0) - 1)
    def _fin():
        out_ref[0, 0] = (rowsum_ref[0]
                         + jnp.sum(jnp.maximum(colmin_ref[...], 0.0))) / N


def kernel(Xc, Xt):
    xc0 = Xc[:, 0:1]
    xc1 = Xc[:, 1:2]

    a01 = (-2.0 * Xc).astype(jnp.bfloat16)
    ones = jnp.ones((N, 3), dtype=jnp.bfloat16)
    zeros = jnp.zeros((N, 3), dtype=jnp.bfloat16)
    A = jnp.concatenate([a01, ones, zeros], axis=1)

    nt = Xt[:, 0] * Xt[:, 0] + Xt[:, 1] * Xt[:, 1]
    m1, m2, m3 = _split3(nt)
    B = jnp.concatenate([
        Xt[:, 0].astype(jnp.bfloat16).reshape(1, N),
        Xt[:, 1].astype(jnp.bfloat16).reshape(1, N),
        m1.astype(jnp.bfloat16).reshape(1, N),
        m2.astype(jnp.bfloat16).reshape(1, N),
        m3.astype(jnp.bfloat16).reshape(1, N),
        jnp.zeros((3, N), dtype=jnp.bfloat16),
    ], axis=0)

    out = pl.pallas_call(
        _body,
        grid=(N // BM,),
        in_specs=[
            pl.BlockSpec((BM, 8), lambda i: (i, 0)),
            pl.BlockSpec((8, N), lambda i: (0, 0)),
            pl.BlockSpec((BM, 1), lambda i: (i, 0)),
            pl.BlockSpec((BM, 1), lambda i: (i, 0)),
        ],
        out_specs=pl.BlockSpec((1, 1), lambda i: (0, 0), memory_space=pltpu.SMEM),
        out_shape=jax.ShapeDtypeStruct((1, 1), jnp.float32),
        scratch_shapes=[
            pltpu.VMEM((1, N), jnp.float32),
            pltpu.SMEM((1,), jnp.float32),
        ],
    )(A, B, xc0, xc1)
    return out[0, 0]


# speedup vs baseline: 2.5014x; 1.4270x over previous
import jax
import jax.numpy as jnp
from jax.experimental import pallas as pl
from jax.experimental.pallas import tpu as pltpu

N = 8192
BM = 512


def _split3(x):
    mask = jnp.uint32(0xFFFF0000)

    def trunc(v):
        return jax.lax.bitcast_convert_type(
            jax.lax.bitcast_convert_type(v, jnp.uint32) & mask, jnp.float32)

    m1 = trunc(x)
    r1 = x - m1
    m2 = trunc(r1)
    r2 = r1 - m2
    m3 = trunc(r2)
    return m1, m2, m3


def _body(a_ref, b_ref, out_ref, colmin_ref, rowsum_ref):
    i = pl.program_id(0)

    @pl.when(i == 0)
    def _init():
        colmin_ref[...] = jnp.full((1, N), jnp.inf, dtype=jnp.float32)
        rowsum_ref[0] = 0.0

    W = jax.lax.dot_general(
        a_ref[...], b_ref[...],
        dimension_numbers=(((1,), (0,)), ((), ())),
        preferred_element_type=jnp.float32,
    )

    rmin = jnp.min(W, axis=1, keepdims=True)
    rowsum_ref[0] += jnp.sum(jnp.maximum(rmin, 0.0))
    colmin_ref[...] = jnp.minimum(colmin_ref[...], jnp.min(W, axis=0, keepdims=True))

    @pl.when(i == pl.num_programs(0) - 1)
    def _fin():
        out_ref[0, 0] = (rowsum_ref[0]
                         + jnp.sum(jnp.maximum(colmin_ref[...], 0.0))) / N


def kernel(Xc, Xt):
    a01 = (-2.0 * Xc).astype(jnp.bfloat16)
    nc = Xc[:, 0] * Xc[:, 0] + Xc[:, 1] * Xc[:, 1]
    n1, n2, n3 = _split3(nc)
    ones_c = jnp.ones((N, 3), dtype=jnp.bfloat16)
    A = jnp.concatenate([
        a01,
        n1.astype(jnp.bfloat16).reshape(N, 1),
        n2.astype(jnp.bfloat16).reshape(N, 1),
        n3.astype(jnp.bfloat16).reshape(N, 1),
        ones_c,
    ], axis=1)

    nt = Xt[:, 0] * Xt[:, 0] + Xt[:, 1] * Xt[:, 1]
    m1, m2, m3 = _split3(nt)
    B = jnp.concatenate([
        Xt[:, 0].astype(jnp.bfloat16).reshape(1, N),
        Xt[:, 1].astype(jnp.bfloat16).reshape(1, N),
        jnp.ones((3, N), dtype=jnp.bfloat16),
        m1.astype(jnp.bfloat16).reshape(1, N),
        m2.astype(jnp.bfloat16).reshape(1, N),
        m3.astype(jnp.bfloat16).reshape(1, N),
    ], axis=0)

    out = pl.pallas_call(
        _body,
        grid=(N // BM,),
        in_specs=[
            pl.BlockSpec((BM, 8), lambda i: (i, 0)),
            pl.BlockSpec((8, N), lambda i: (0, 0)),
        ],
        out_specs=pl.BlockSpec((1, 1), lambda i: (0, 0), memory_space=pltpu.SMEM),
        out_shape=jax.ShapeDtypeStruct((1, 1), jnp.float32),
        scratch_shapes=[
            pltpu.VMEM((1, N), jnp.float32),
            pltpu.SMEM((1,), jnp.float32),
        ],
    )(A, B)
    return out[0, 0]


# speedup vs baseline: 2.6623x; 1.0643x over previous
import jax
import jax.numpy as jnp
from jax.experimental import pallas as pl
from jax.experimental.pallas import tpu as pltpu

N = 8192
BM = 1024


def _split3(x):
    mask = jnp.uint32(0xFFFF0000)

    def trunc(v):
        return jax.lax.bitcast_convert_type(
            jax.lax.bitcast_convert_type(v, jnp.uint32) & mask, jnp.float32)

    m1 = trunc(x)
    r1 = x - m1
    m2 = trunc(r1)
    r2 = r1 - m2
    m3 = trunc(r2)
    return m1, m2, m3


def _body(a_ref, b_ref, out_ref, colmin_ref, rowsum_ref):
    i = pl.program_id(0)

    @pl.when(i == 0)
    def _init():
        colmin_ref[...] = jnp.full((1, N), jnp.inf, dtype=jnp.float32)
        rowsum_ref[0] = 0.0

    W = jax.lax.dot_general(
        a_ref[...], b_ref[...],
        dimension_numbers=(((1,), (0,)), ((), ())),
        preferred_element_type=jnp.float32,
    )

    rmin = jnp.min(W, axis=1, keepdims=True)
    rowsum_ref[0] += jnp.sum(jnp.maximum(rmin, 0.0))
    colmin_ref[...] = jnp.minimum(colmin_ref[...], jnp.min(W, axis=0, keepdims=True))

    @pl.when(i == pl.num_programs(0) - 1)
    def _fin():
        out_ref[0, 0] = (rowsum_ref[0]
                         + jnp.sum(jnp.maximum(colmin_ref[...], 0.0))) / N


def kernel(Xc, Xt):
    a01 = (-2.0 * Xc).astype(jnp.bfloat16)
    nc = Xc[:, 0] * Xc[:, 0] + Xc[:, 1] * Xc[:, 1]
    n1, n2, n3 = _split3(nc)
    ones_c = jnp.ones((N, 3), dtype=jnp.bfloat16)
    A = jnp.concatenate([
        a01,
        n1.astype(jnp.bfloat16).reshape(N, 1),
        n2.astype(jnp.bfloat16).reshape(N, 1),
        n3.astype(jnp.bfloat16).reshape(N, 1),
        ones_c,
    ], axis=1)

    nt = Xt[:, 0] * Xt[:, 0] + Xt[:, 1] * Xt[:, 1]
    m1, m2, m3 = _split3(nt)
    B = jnp.concatenate([
        Xt[:, 0].astype(jnp.bfloat16).reshape(1, N),
        Xt[:, 1].astype(jnp.bfloat16).reshape(1, N),
        jnp.ones((3, N), dtype=jnp.bfloat16),
        m1.astype(jnp.bfloat16).reshape(1, N),
        m2.astype(jnp.bfloat16).reshape(1, N),
        m3.astype(jnp.bfloat16).reshape(1, N),
    ], axis=0)

    out = pl.pallas_call(
        _body,
        grid=(N // BM,),
        in_specs=[
            pl.BlockSpec((BM, 8), lambda i: (i, 0)),
            pl.BlockSpec((8, N), lambda i: (0, 0)),
        ],
        out_specs=pl.BlockSpec((1, 1), lambda i: (0, 0), memory_space=pltpu.SMEM),
        out_shape=jax.ShapeDtypeStruct((1, 1), jnp.float32),
        scratch_shapes=[
            pltpu.VMEM((1, N), jnp.float32),
            pltpu.SMEM((1,), jnp.float32),
        ],
    )(A, B)
    return out[0, 0]
